# initial kernel scaffold (unmeasured)
import jax
import jax.numpy as jnp
from jax import lax
from jax.experimental import pallas as pl
from jax.experimental.pallas import tpu as pltpu

B = 32
H = 16
D = 128
BS = 32
NB = 256
CHUNK = 16
SCALE = D ** -0.5


def kernel(Q, K, V, bt, lens):
    n_pages = K.shape[0]
    n_chunks = n_pages // CHUNK
    lens2 = lens.reshape(B, 1)

    def body(bt_ref, lens_ref, q_ref, k_ref, v_ref, out_ref,
             acc_o, acc_l, recv_o, recv_l, send_sems, recv_sems):
        j = pl.program_id(0)
        my_x = lax.axis_index("x")
        my_y = lax.axis_index("y")
        my_z = lax.axis_index("z")
        partner = (my_x, 1 - my_y, my_z)

        @pl.when(j == 0)
        def _init():
            acc_o[...] = jnp.zeros_like(acc_o)
            acc_l[...] = jnp.zeros_like(acc_l)
            barrier_sem = pltpu.get_barrier_semaphore()
            pl.semaphore_signal(
                barrier_sem, inc=1, device_id=partner,
                device_id_type=pl.DeviceIdType.MESH,
            )
            pl.semaphore_wait(barrier_sem, 1)

        page_ids = (my_y * n_pages + j * CHUNK
                    + lax.broadcasted_iota(jnp.int32, (1, 1, CHUNK), 2))
        btv = bt_ref[...]
        col = lax.broadcasted_iota(jnp.int32, (B, NB, 1), 1)
        hit = (btv[:, :, None] == page_ids) & (col < lens_ref[...][:, :, None])
        counts = jnp.sum(hit.astype(jnp.float32), axis=1)
        counts_k = jnp.broadcast_to(
            counts[:, :, None], (B, CHUNK, BS)).reshape(B, CHUNK * BS)

        q = q_ref[...].reshape(B, H, D).astype(jnp.bfloat16)
        k = k_ref[...].reshape(CHUNK * BS, H, D).astype(jnp.bfloat16)
        s = lax.dot_general(
            q, k, (((2,), (2,)), ((1,), (1,))),
            preferred_element_type=jnp.float32)
        e = jnp.exp(s * SCALE) * counts_k[None, :, :]
        acc_l[...] += jnp.sum(e, axis=2)
        v = v_ref[...].reshape(CHUNK * BS, H, D).astype(jnp.bfloat16)
        acc_o[...] += lax.dot_general(
            e.astype(jnp.bfloat16), v, (((2,), (0,)), ((0,), (1,))),
            preferred_element_type=jnp.float32)

        @pl.when(j == n_chunks - 1)
        def _finish():
            rdma_o = pltpu.make_async_remote_copy(
                src_ref=acc_o, dst_ref=recv_o,
                send_sem=send_sems.at[0], recv_sem=recv_sems.at[0],
                device_id=partner, device_id_type=pl.DeviceIdType.MESH,
            )
            rdma_l = pltpu.make_async_remote_copy(
                src_ref=acc_l, dst_ref=recv_l,
                send_sem=send_sems.at[1], recv_sem=recv_sems.at[1],
                device_id=partner, device_id_type=pl.DeviceIdType.MESH,
            )
            rdma_o.start()
            rdma_l.start()
            rdma_o.wait()
            rdma_l.wait()
            tot_o = acc_o[...] + recv_o[...]
            tot_l = acc_l[...] + recv_l[...]
            res = tot_o / tot_l[:, :, None]
            out_ref[...] = jnp.transpose(res, (1, 0, 2)).reshape(B, 1, H, D)

    return pl.pallas_call(
        body,
        grid=(n_chunks,),
        in_specs=[
            pl.BlockSpec((B, NB), lambda j: (0, 0)),
            pl.BlockSpec((B, 1), lambda j: (0, 0)),
            pl.BlockSpec((B, 1, H, D), lambda j: (0, 0, 0, 0)),
            pl.BlockSpec((CHUNK, BS, H, D), lambda j: (j, 0, 0, 0)),
            pl.BlockSpec((CHUNK, BS, H, D), lambda j: (j, 0, 0, 0)),
        ],
        out_specs=pl.BlockSpec((B, 1, H, D), lambda j: (0, 0, 0, 0)),
        out_shape=jax.ShapeDtypeStruct((B, 1, H, D), jnp.float32),
        scratch_shapes=[
            pltpu.VMEM((H, B, D), jnp.float32),
            pltpu.VMEM((H, B), jnp.float32),
            pltpu.VMEM((H, B, D), jnp.float32),
            pltpu.VMEM((H, B), jnp.float32),
            pltpu.SemaphoreType.DMA((2,)),
            pltpu.SemaphoreType.DMA((2,)),
        ],
        compiler_params=pltpu.CompilerParams(
            dimension_semantics=("arbitrary",),
            collective_id=0,
        ),
    )(bt, lens2, Q, K, V)


# baseline (device time: 222471 ns/iter reference)
import jax
import jax.numpy as jnp
from jax import lax
from jax.experimental import pallas as pl
from jax.experimental.pallas import tpu as pltpu

B = 32
H = 16
D = 128
BS = 32
NB = 256
CHUNK = 16
SCALE = D ** -0.5


def kernel(Q, K, V, bt, lens):
    n_pages = K.shape[0]
    n_chunks = n_pages // CHUNK
    lens2 = lens.reshape(B, 1)

    def body(bt_ref, lens_ref, q_ref, k_ref, v_ref, out_ref,
             acc_o, acc_l, recv_o, recv_l, send_sems, recv_sems):
        j = pl.program_id(0)
        my_x = lax.axis_index("x")
        my_y = lax.axis_index("y")
        my_z = lax.axis_index("z")
        partner = (my_x, 1 - my_y, my_z)

        @pl.when(j == 0)
        def _init():
            acc_o[...] = jnp.zeros_like(acc_o)
            acc_l[...] = jnp.zeros_like(acc_l)
            barrier_sem = pltpu.get_barrier_semaphore()
            pl.semaphore_signal(
                barrier_sem, inc=1, device_id=partner,
                device_id_type=pl.DeviceIdType.MESH,
            )
            pl.semaphore_wait(barrier_sem, 1)

        page_ids = (my_y * n_pages + j * CHUNK
                    + lax.broadcasted_iota(jnp.int32, (1, 1, CHUNK), 2))
        btv = bt_ref[...]
        col = lax.broadcasted_iota(jnp.int32, (B, NB, 1), 1)
        hit = (btv[:, :, None] == page_ids) & (col < lens_ref[...][:, :, None])
        counts = jnp.sum(hit.astype(jnp.float32), axis=1)
        counts_k = jnp.broadcast_to(
            counts[:, :, None], (B, CHUNK, BS)).reshape(B, CHUNK * BS)

        q = q_ref[...].reshape(B, H, D).astype(jnp.bfloat16)
        k = k_ref[...].reshape(CHUNK * BS, H, D).astype(jnp.bfloat16)
        s = lax.dot_general(
            q, k, (((2,), (2,)), ((1,), (1,))),
            preferred_element_type=jnp.float32)
        e = jnp.exp(s * SCALE) * counts_k[None, :, :]
        acc_l[...] += jnp.sum(e, axis=2)
        v = v_ref[...].reshape(CHUNK * BS, H, D).astype(jnp.bfloat16)
        acc_o[...] += lax.dot_general(
            e.astype(jnp.bfloat16), v, (((2,), (0,)), ((0,), (1,))),
            preferred_element_type=jnp.float32)

        @pl.when(j == n_chunks - 1)
        def _finish():
            rdma_o = pltpu.make_async_remote_copy(
                src_ref=acc_o, dst_ref=recv_o,
                send_sem=send_sems.at[0], recv_sem=recv_sems.at[0],
                device_id=partner, device_id_type=pl.DeviceIdType.MESH,
            )
            rdma_l = pltpu.make_async_remote_copy(
                src_ref=acc_l, dst_ref=recv_l,
                send_sem=send_sems.at[1], recv_sem=recv_sems.at[1],
                device_id=partner, device_id_type=pl.DeviceIdType.MESH,
            )
            rdma_o.start()
            rdma_l.start()
            rdma_o.wait()
            rdma_l.wait()
            tot_o = acc_o[...] + recv_o[...]
            tot_l = acc_l[...] + recv_l[...]
            res = tot_o / tot_l[:, :, None]
            out_ref[...] = jnp.transpose(res, (1, 0, 2)).reshape(B, 1, H, D)

    return pl.pallas_call(
        body,
        grid=(n_chunks,),
        in_specs=[
            pl.BlockSpec((B, NB), lambda j: (0, 0)),
            pl.BlockSpec((B, 1), lambda j: (0, 0)),
            pl.BlockSpec((B, 1, H, D), lambda j: (0, 0, 0, 0)),
            pl.BlockSpec((CHUNK, BS, H, D), lambda j: (j, 0, 0, 0)),
            pl.BlockSpec((CHUNK, BS, H, D), lambda j: (j, 0, 0, 0)),
        ],
        out_specs=pl.BlockSpec((B, 1, H, D), lambda j: (0, 0, 0, 0)),
        out_shape=jax.ShapeDtypeStruct((B, 1, H, D), jnp.float32),
        scratch_shapes=[
            pltpu.VMEM((H, B, D), jnp.float32),
            pltpu.VMEM((H, B), jnp.float32),
            pltpu.VMEM((H, B, D), jnp.float32),
            pltpu.VMEM((H, B), jnp.float32),
            pltpu.SemaphoreType.DMA((2,)),
            pltpu.SemaphoreType.DMA((2,)),
        ],
        compiler_params=pltpu.CompilerParams(
            dimension_semantics=("arbitrary",),
            collective_id=0,
            vmem_limit_bytes=100 * 1024 * 1024,
        ),
    )(bt, lens2, Q, K, V)


# device time: 88295 ns/iter; 2.5196x vs baseline; 2.5196x over previous
import jax
import jax.numpy as jnp
from jax import lax
from jax.experimental import pallas as pl
from jax.experimental.pallas import tpu as pltpu

B = 32
H = 16
D = 128
BS = 32
NB = 256
CHUNK = 8
QUAD = 4
SCALE = D ** -0.5


def kernel(Q, K, V, bt, lens):
    n_pages = K.shape[0]
    my_pages = n_pages // QUAD
    n_steps = my_pages // CHUNK
    lens2 = lens.reshape(B, 1)

    x = lax.axis_index("x")
    z = lax.axis_index("z")
    chunk_base = ((x * 2 + z) * n_steps).astype(jnp.int32).reshape(1)

    def body(cb_ref, bt_ref, lens_ref, q_ref, k_ref, v_ref, out_ref,
             acc_o, acc_l, recv_o, recv_l, send_sems, recv_sems):
        j = pl.program_id(0)
        my_x = lax.axis_index("x")
        my_y = lax.axis_index("y")
        my_z = lax.axis_index("z")
        partners = [
            (my_x, 1 - my_y, my_z),
            (1 - my_x, my_y, my_z),
            (my_x, my_y, 1 - my_z),
        ]

        @pl.when(j == 0)
        def _init():
            acc_o[...] = jnp.zeros_like(acc_o)
            acc_l[...] = jnp.zeros_like(acc_l)
            barrier_sem = pltpu.get_barrier_semaphore()
            for p in partners:
                pl.semaphore_signal(
                    barrier_sem, inc=1, device_id=p,
                    device_id_type=pl.DeviceIdType.MESH,
                )
            pl.semaphore_wait(barrier_sem, 3)

        page_ids = (my_y * n_pages + (cb_ref[0] + j) * CHUNK
                    + lax.broadcasted_iota(jnp.int32, (1, 1, CHUNK), 2))
        btv = bt_ref[...]
        col = lax.broadcasted_iota(jnp.int32, (B, NB, 1), 1)
        hit = (btv[:, :, None] == page_ids) & (col < lens_ref[...][:, :, None])
        counts = jnp.sum(hit.astype(jnp.float32), axis=1)
        counts_k = jnp.broadcast_to(
            counts[:, :, None], (B, CHUNK, BS)).reshape(B, CHUNK * BS)

        q = q_ref[...].reshape(B, H, D).astype(jnp.bfloat16)
        k = k_ref[...].reshape(CHUNK * BS, H, D).astype(jnp.bfloat16)
        s = lax.dot_general(
            q, k, (((2,), (2,)), ((1,), (1,))),
            preferred_element_type=jnp.float32)
        e = jnp.exp(s * SCALE) * counts_k[None, :, :]
        acc_l[...] += jnp.sum(e, axis=2)
        v = v_ref[...].reshape(CHUNK * BS, H, D).astype(jnp.bfloat16)
        acc_o[...] += lax.dot_general(
            e.astype(jnp.bfloat16), v, (((2,), (0,)), ((0,), (1,))),
            preferred_element_type=jnp.float32)

        @pl.when(j == n_steps - 1)
        def _finish():
            for st, p in enumerate(partners):
                rdma_o = pltpu.make_async_remote_copy(
                    src_ref=acc_o, dst_ref=recv_o.at[st],
                    send_sem=send_sems.at[st, 0], recv_sem=recv_sems.at[st, 0],
                    device_id=p, device_id_type=pl.DeviceIdType.MESH,
                )
                rdma_l = pltpu.make_async_remote_copy(
                    src_ref=acc_l, dst_ref=recv_l.at[st],
                    send_sem=send_sems.at[st, 1], recv_sem=recv_sems.at[st, 1],
                    device_id=p, device_id_type=pl.DeviceIdType.MESH,
                )
                rdma_o.start()
                rdma_l.start()
                rdma_o.wait()
                rdma_l.wait()
                acc_o[...] += recv_o[st]
                acc_l[...] += recv_l[st]
            res = acc_o[...] / acc_l[...][:, :, None]
            out_ref[...] = jnp.transpose(res, (1, 0, 2)).reshape(B, 1, H, D)

    grid_spec = pltpu.PrefetchScalarGridSpec(
        num_scalar_prefetch=1,
        grid=(n_steps,),
        in_specs=[
            pl.BlockSpec((B, NB), lambda j, cb: (0, 0)),
            pl.BlockSpec((B, 1), lambda j, cb: (0, 0)),
            pl.BlockSpec((B, 1, H, D), lambda j, cb: (0, 0, 0, 0)),
            pl.BlockSpec((CHUNK, BS, H, D), lambda j, cb: (cb[0] + j, 0, 0, 0)),
            pl.BlockSpec((CHUNK, BS, H, D), lambda j, cb: (cb[0] + j, 0, 0, 0)),
        ],
        out_specs=pl.BlockSpec((B, 1, H, D), lambda j, cb: (0, 0, 0, 0)),
        scratch_shapes=[
            pltpu.VMEM((H, B, D), jnp.float32),
            pltpu.VMEM((H, B), jnp.float32),
            pltpu.VMEM((3, H, B, D), jnp.float32),
            pltpu.VMEM((3, H, B), jnp.float32),
            pltpu.SemaphoreType.DMA((3, 2)),
            pltpu.SemaphoreType.DMA((3, 2)),
        ],
    )

    return pl.pallas_call(
        body,
        grid_spec=grid_spec,
        out_shape=jax.ShapeDtypeStruct((B, 1, H, D), jnp.float32),
        compiler_params=pltpu.CompilerParams(
            dimension_semantics=("arbitrary",),
            collective_id=0,
            vmem_limit_bytes=100 * 1024 * 1024,
        ),
    )(chunk_base, bt, lens2, Q, K, V)


# device time: 65246 ns/iter; 3.4097x vs baseline; 1.3533x over previous
import jax
import jax.numpy as jnp
from jax import lax
from jax.experimental import pallas as pl
from jax.experimental.pallas import tpu as pltpu

B = 32
H = 16
D = 128
BS = 32
NB = 256
CHUNK = 16
QUAD = 4
SCALE = D ** -0.5


def kernel(Q, K, V, bt, lens):
    n_pages = K.shape[0]
    my_pages = n_pages // QUAD
    n_steps = my_pages // CHUNK
    lens2 = lens.reshape(B, 1)

    x = lax.axis_index("x")
    z = lax.axis_index("z")
    chunk_base = ((x * 2 + z) * n_steps).astype(jnp.int32).reshape(1)

    def body(cb_ref, bt_ref, lens_ref, q_ref, k_ref, v_ref, out_ref,
             acc_o, acc_l, send_o, recv_o, recv_l, send_sems, recv_sems):
        j = pl.program_id(0)
        my_x = lax.axis_index("x")
        my_y = lax.axis_index("y")
        my_z = lax.axis_index("z")
        partners = [
            (my_x, 1 - my_y, my_z),
            (1 - my_x, my_y, my_z),
            (my_x, my_y, 1 - my_z),
        ]

        @pl.when(j == 0)
        def _init():
            acc_o[...] = jnp.zeros_like(acc_o)
            acc_l[...] = jnp.zeros_like(acc_l)
            barrier_sem = pltpu.get_barrier_semaphore()
            for p in partners:
                pl.semaphore_signal(
                    barrier_sem, inc=1, device_id=p,
                    device_id_type=pl.DeviceIdType.MESH,
                )
            pl.semaphore_wait(barrier_sem, 3)

        page_ids = (my_y * n_pages + (cb_ref[0] + j) * CHUNK
                    + lax.broadcasted_iota(jnp.int32, (1, CHUNK, 1), 1))
        btv = bt_ref[...]
        col = lax.broadcasted_iota(jnp.int32, (1, 1, NB), 2)
        hit = (btv[:, None, :] == page_ids) & (col < lens_ref[...][:, :, None])
        counts = jnp.sum(hit.astype(jnp.float32), axis=2)
        counts_k = jnp.broadcast_to(
            counts[:, :, None], (B, CHUNK, BS)).reshape(B, CHUNK * BS)

        q = (q_ref[...] * SCALE).reshape(B, H, D).astype(jnp.bfloat16)
        k = k_ref[...].reshape(CHUNK * BS, H, D).astype(jnp.bfloat16)
        s = lax.dot_general(
            q, k, (((2,), (2,)), ((1,), (1,))),
            preferred_element_type=jnp.float32)
        e = jnp.exp(s) * counts_k[None, :, :]
        acc_l[...] += jnp.sum(e, axis=2)
        v = v_ref[...].reshape(CHUNK * BS, H, D).astype(jnp.bfloat16)
        acc_o[...] += lax.dot_general(
            e.astype(jnp.bfloat16), v, (((2,), (0,)), ((0,), (1,))),
            preferred_element_type=jnp.float32)

        @pl.when(j == n_steps - 1)
        def _finish():
            for st, p in enumerate(partners):
                send_o[...] = acc_o[...].astype(jnp.bfloat16)
                rdma_o = pltpu.make_async_remote_copy(
                    src_ref=send_o, dst_ref=recv_o.at[st],
                    send_sem=send_sems.at[st, 0], recv_sem=recv_sems.at[st, 0],
                    device_id=p, device_id_type=pl.DeviceIdType.MESH,
                )
                rdma_l = pltpu.make_async_remote_copy(
                    src_ref=acc_l, dst_ref=recv_l.at[st],
                    send_sem=send_sems.at[st, 1], recv_sem=recv_sems.at[st, 1],
                    device_id=p, device_id_type=pl.DeviceIdType.MESH,
                )
                rdma_o.start()
                rdma_l.start()
                rdma_o.wait()
                rdma_l.wait()
                acc_o[...] += recv_o[st].astype(jnp.float32)
                acc_l[...] += recv_l[st]
            res = acc_o[...] / acc_l[...][:, :, None]
            out_ref[...] = jnp.transpose(res, (1, 0, 2)).reshape(B, 1, H, D)

    grid_spec = pltpu.PrefetchScalarGridSpec(
        num_scalar_prefetch=1,
        grid=(n_steps,),
        in_specs=[
            pl.BlockSpec((B, NB), lambda j, cb: (0, 0)),
            pl.BlockSpec((B, 1), lambda j, cb: (0, 0)),
            pl.BlockSpec((B, 1, H, D), lambda j, cb: (0, 0, 0, 0)),
            pl.BlockSpec((CHUNK, BS, H, D), lambda j, cb: (cb[0] + j, 0, 0, 0)),
            pl.BlockSpec((CHUNK, BS, H, D), lambda j, cb: (cb[0] + j, 0, 0, 0)),
        ],
        out_specs=pl.BlockSpec((B, 1, H, D), lambda j, cb: (0, 0, 0, 0)),
        scratch_shapes=[
            pltpu.VMEM((H, B, D), jnp.float32),
            pltpu.VMEM((H, B), jnp.float32),
            pltpu.VMEM((H, B, D), jnp.bfloat16),
            pltpu.VMEM((3, H, B, D), jnp.bfloat16),
            pltpu.VMEM((3, H, B), jnp.float32),
            pltpu.SemaphoreType.DMA((3, 2)),
            pltpu.SemaphoreType.DMA((3, 2)),
        ],
    )

    return pl.pallas_call(
        body,
        grid_spec=grid_spec,
        out_shape=jax.ShapeDtypeStruct((B, 1, H, D), jnp.float32),
        compiler_params=pltpu.CompilerParams(
            dimension_semantics=("arbitrary",),
            collective_id=0,
            vmem_limit_bytes=100 * 1024 * 1024,
        ),
    )(chunk_base, bt, lens2, Q, K, V)
